# baseline (speedup 1.0000x reference)
import jax
import jax.numpy as jnp
from jax.experimental import pallas as pl
from jax.experimental.pallas import tpu as pltpu

_EMB = 64
_POS = 588
_DIN = 16
_B = 1024
_NCH = 8

_G = 4
_PG = _POS // _G
_KP = _G * _DIN
_NP = _G * _EMB

_BB = 128


def _kernel_body(cb_ref, emb_ref, w_ref, x_ref, o_ref, y_scr):
    @pl.when(pl.program_id(0) == 0)
    def _init():
        iota = jax.lax.broadcasted_iota(jnp.int32, (_PG, _NCH), 1)
        oh = jnp.concatenate(
            [(cb_ref[:, g][:, None] == iota).astype(jnp.float32)
             for g in range(_G)], axis=1)
        y_scr[...] = jnp.dot(oh, emb_ref[...],
                             preferred_element_type=jnp.float32)

    w = w_ref[...]
    y = y_scr[...]
    for bb in range(_BB):
        d = jnp.dot(x_ref[bb], w, preferred_element_type=jnp.float32)
        o_ref[bb] = (d + y).astype(jnp.bfloat16)


_NSPLIT = 2
_BH = _B // _NSPLIT


def kernel(x, emb_table, W, b, channel_base):
    eye = jnp.eye(_G, dtype=jnp.float32)
    Wg = jnp.kron(eye, W).astype(jnp.bfloat16)
    embg = jnp.kron(eye, emb_table + b[None, :])
    cbg = channel_base.astype(jnp.int32).reshape(_PG, _G)
    halves = []
    for s in range(_NSPLIT):
        xg = x[s * _BH:(s + 1) * _BH].reshape(_BH, _PG, _KP)
        xg = xg.astype(jnp.bfloat16)
        out = pl.pallas_call(
            _kernel_body,
            grid=(_BH // _BB,),
            in_specs=[
                pl.BlockSpec((_PG, _G), lambda i: (0, 0)),
                pl.BlockSpec((_G * _NCH, _NP), lambda i: (0, 0)),
                pl.BlockSpec((_KP, _NP), lambda i: (0, 0)),
                pl.BlockSpec((_BB, _PG, _KP), lambda i: (i, 0, 0)),
            ],
            out_specs=pl.BlockSpec((_BB, _PG, _NP), lambda i: (i, 0, 0)),
            out_shape=jax.ShapeDtypeStruct((_BH, _PG, _NP), jnp.bfloat16),
            scratch_shapes=[pltpu.VMEM((_PG, _NP), jnp.float32)],
        )(cbg, embg, Wg, xg)
        halves.append(out.astype(jnp.float32).reshape(_BH, _POS, _EMB))
    return jnp.concatenate(halves, axis=0)


# speedup vs baseline: 1.2878x; 1.2878x over previous
import jax
import jax.numpy as jnp
from jax.experimental import pallas as pl
from jax.experimental.pallas import tpu as pltpu

_EMB = 64
_POS = 588
_DIN = 16
_B = 1024
_NCH = 8

_G = 4
_PG = _POS // _G
_KP = _G * _DIN
_NP = _G * _EMB

_BB = 128


def _kernel_body(cb_ref, emb_ref, w_ref, x_ref, o_ref, y_scr):
    @pl.when(pl.program_id(0) == 0)
    def _init():
        iota = jax.lax.broadcasted_iota(jnp.int32, (_PG, _NCH), 1)
        oh = jnp.concatenate(
            [(cb_ref[:, g][:, None] == iota).astype(jnp.float32)
             for g in range(_G)], axis=1)
        y_scr[...] = jnp.dot(oh, emb_ref[...],
                             preferred_element_type=jnp.float32)

    w = w_ref[...]
    y = y_scr[...]
    for bb in range(_BB):
        d = jnp.dot(x_ref[bb], w, preferred_element_type=jnp.float32)
        o_ref[bb] = (d + y).astype(jnp.bfloat16)


def kernel(x, emb_table, W, b, channel_base):
    xg = x.reshape(_B, _PG, _KP).astype(jnp.bfloat16)
    eye = jnp.eye(_G, dtype=jnp.float32)
    Wg = jnp.kron(eye, W).astype(jnp.bfloat16)
    embg = jnp.kron(eye, emb_table + b[None, :])
    cbg = channel_base.astype(jnp.int32).reshape(_PG, _G)
    out = pl.pallas_call(
        _kernel_body,
        grid=(_B // _BB,),
        in_specs=[
            pl.BlockSpec((_PG, _G), lambda i: (0, 0)),
            pl.BlockSpec((_G * _NCH, _NP), lambda i: (0, 0)),
            pl.BlockSpec((_KP, _NP), lambda i: (0, 0)),
            pl.BlockSpec((_BB, _PG, _KP), lambda i: (i, 0, 0)),
        ],
        out_specs=pl.BlockSpec((_BB, _PG, _NP), lambda i: (i, 0, 0)),
        out_shape=jax.ShapeDtypeStruct((_B, _PG, _NP), jnp.bfloat16),
        scratch_shapes=[pltpu.VMEM((_PG, _NP), jnp.float32)],
    )(cbg, embg, Wg, xg)
    return out.astype(jnp.float32).reshape(_B, _POS, _EMB)
